# baseline (speedup 1.0000x reference)
import functools

import jax
import jax.numpy as jnp
from jax.experimental import pallas as pl
from jax.experimental.pallas import tpu as pltpu

_N = 10000
_G = 64
_H = 4
_C = 256


def _mm_kernel(x_ref, w_ref, b_ref, o_ref):
    o_ref[...] = (
        jnp.dot(x_ref[...], w_ref[...], preferred_element_type=jnp.float32)
        + b_ref[...]
    )


def _mm(x, w, b, bm=1000, bn=1024):
    M, K = x.shape
    N = w.shape[1]
    bm = min(bm, M)
    bn = min(bn, N)
    assert M % bm == 0 and N % bn == 0
    b2 = b.reshape(1, N)
    return pl.pallas_call(
        _mm_kernel,
        grid=(M // bm, N // bn),
        in_specs=[
            pl.BlockSpec((bm, K), lambda i, j: (i, 0)),
            pl.BlockSpec((K, bn), lambda i, j: (0, j)),
            pl.BlockSpec((1, bn), lambda i, j: (0, j)),
        ],
        out_specs=pl.BlockSpec((bm, bn), lambda i, j: (i, j)),
        out_shape=jax.ShapeDtypeStruct((M, N), jnp.float32),
    )(x, w, b2)


def _segment_softmax(scores, seg, num_segments):
    m = jax.ops.segment_max(scores, seg, num_segments=num_segments)
    m = jnp.where(jnp.isfinite(m), m, 0.0)
    e = jnp.exp(scores - m[seg])
    s = jax.ops.segment_sum(e, seg, num_segments=num_segments)
    return e / (s[seg] + 1e-16)


def _gat_layer(x, edge_index, edge_attr, Wl, bl, Wr, br, We, att, bias, concat):
    n = x.shape[0]
    src0 = edge_index[0]
    dst0 = edge_index[1]
    deg = jax.ops.segment_sum(
        jnp.ones((src0.shape[0],), jnp.float32), dst0, num_segments=n
    )
    self_attr = (
        jax.ops.segment_sum(edge_attr, dst0, num_segments=n)
        / jnp.maximum(deg, 1.0)[:, None]
    )
    loop = jnp.arange(n, dtype=src0.dtype)
    src = jnp.concatenate([src0, loop])
    dst = jnp.concatenate([dst0, loop])
    ea = jnp.concatenate([edge_attr, self_attr], axis=0)

    Wlr = jnp.concatenate([Wl, Wr], axis=1)
    blr = jnp.concatenate([bl, br], axis=0)
    xlr = _mm(x, Wlr, blr)
    xl = xlr[:, : _H * _C].reshape(n, _H, _C)
    xr = xlr[:, _H * _C :].reshape(n, _H, _C)

    ee = (ea @ We).reshape(-1, _H, _C)
    m = xl[src] + xr[dst] + ee
    m = jax.nn.leaky_relu(m, 0.2)
    alpha = jnp.sum(m * att[None, :, :], axis=-1)
    alpha = _segment_softmax(alpha, dst, n)
    msg = xl[src] * alpha[:, :, None]
    out = jax.ops.segment_sum(msg, dst, num_segments=n)
    if concat:
        out = out.reshape(n, _H * _C)
    else:
        out = out.mean(axis=1)
    return out + bias


def _conv_head(h, conv1_w, conv1_b, conv2_w, conv2_b):
    hs = h[:, None, :]
    y = jax.lax.conv_general_dilated(
        hs, conv1_w, window_strides=(1,), padding=((1, 1),),
        dimension_numbers=("NCH", "OIH", "NCH"),
    ) + conv1_b[None, :, None]
    y = jax.nn.relu(y)
    n, c, l = y.shape
    y = y.reshape(n, c, l // 2, 2).max(axis=-1)
    y = jax.lax.conv_general_dilated(
        y, conv2_w, window_strides=(1,), padding=((1, 1),),
        dimension_numbers=("NCH", "OIH", "NCH"),
    ) + conv2_b[None, :, None]
    y = jax.nn.relu(y)
    n, c, l = y.shape
    y = y.reshape(n, c, l // 2, 2).max(axis=-1)
    return y.reshape(n, -1)


def kernel(x, edge_index, edge_attr, batch, W_l1, b_l1, W_r1, b_r1, W_e1,
           att1, bias1, W_l2, b_l2, W_r2, b_r2, W_e2, att2, bias2, conv1_w,
           conv1_b, conv2_w, conv2_b, fc_seq_w, fc_seq_b, fc_w, fc_b):
    h = jax.nn.relu(
        _gat_layer(x, edge_index, edge_attr, W_l1, b_l1, W_r1, b_r1, W_e1,
                   att1, bias1, True)
    )
    h = jax.nn.relu(
        _gat_layer(h, edge_index, edge_attr, W_l2, b_l2, W_r2, b_r2, W_e2,
                   att2, bias2, False)
    )
    counts = jax.ops.segment_sum(jnp.ones((_N,), jnp.float32), batch,
                                 num_segments=_G)
    denom = jnp.maximum(counts, 1.0)[:, None]
    mean_pool = jax.ops.segment_sum(h, batch, num_segments=_G) / denom
    max_pool = jax.ops.segment_max(h, batch, num_segments=_G)

    flat = _conv_head(h, conv1_w, conv1_b, conv2_w, conv2_b)
    flat_pool = jax.ops.segment_sum(flat, batch, num_segments=_G) / denom
    hs_graph = _mm(flat_pool, fc_seq_w, jnp.zeros((fc_seq_w.shape[1],),
                                                  jnp.float32), bm=64)
    hs_graph = hs_graph + (counts / denom[:, 0])[:, None] * fc_seq_b[None, :]

    gf = jnp.concatenate([mean_pool, max_pool, hs_graph], axis=-1)
    return _mm(gf, fc_w, fc_b, bm=64, bn=256)
